# baseline (speedup 1.0000x reference)
import functools

import jax
import jax.numpy as jnp
from jax import lax
from jax.experimental import pallas as pl
from jax.experimental.pallas import tpu as pltpu, tpu_sc as plsc

_NBOX = 100
_GSTRIDE = 8
_LN2 = 0.6931471805599453


def _log_poly(x):
    bits = lax.bitcast_convert_type(x, jnp.int32)
    e = ((bits >> 23) & 0xFF) - 127
    m = lax.bitcast_convert_type(
        (bits & 0x7FFFFF) | 0x3F800000, jnp.float32)
    s = (m - 1.0) / (m + 1.0)
    s2 = s * s
    p = 1.0 / 11.0
    for c in (1.0 / 9.0, 1.0 / 7.0, 1.0 / 5.0, 1.0 / 3.0, 1.0):
        p = p * s2 + c
    return e.astype(jnp.float32) * _LN2 + 2.0 * s * p


def _sc_call(aT, gt_flat, B, A_pad):
    info = plsc.get_sparse_core_info()
    NC, NS = info.num_cores, info.num_subcores
    NW = NC * NS
    APW = A_pad // NW
    ngrp = APW // 16
    glen = gt_flat.shape[1]

    mesh = plsc.VectorSubcoreMesh(core_axis_name="c", subcore_axis_name="s")

    @functools.partial(
        pl.kernel, mesh=mesh,
        out_type=jax.ShapeDtypeStruct((B, 8, A_pad), jnp.float32),
        scratch_types=[
            pltpu.VMEM((8, APW), jnp.float32),
            pltpu.VMEM((glen, 16), jnp.float32),
            pltpu.VMEM((8, APW), jnp.float32),
            pltpu.SemaphoreType.DMA,
        ],
    )
    def k(aT_hbm, gt_hbm, out_hbm, a_v, g_v, o_v, sem):
        wid = lax.axis_index("s") * NC + lax.axis_index("c")
        base = wid * APW
        pltpu.sync_copy(aT_hbm.at[:, pl.ds(base, APW)], a_v)

        def one_batch(b):
            pltpu.sync_copy(gt_hbm.at[b], g_v)

            def one_group(t):
                off = t * 16
                ax1 = a_v[0, pl.ds(off, 16)]
                ay1 = a_v[1, pl.ds(off, 16)]
                ax2 = a_v[2, pl.ds(off, 16)]
                ay2 = a_v[3, pl.ds(off, 16)]
                aw = a_v[4, pl.ds(off, 16)]
                ah = a_v[5, pl.ds(off, 16)]
                acx = a_v[6, pl.ds(off, 16)]
                acy = a_v[7, pl.ds(off, 16)]
                area_a = aw * ah

                def one_gt(j, carry):
                    best, mx1, my1, mx2, my2, mcls = carry
                    r = j * _GSTRIDE
                    bx1 = g_v[r]
                    by1 = g_v[r + 1]
                    bx2 = g_v[r + 2]
                    by2 = g_v[r + 3]
                    barea = g_v[r + 4]
                    bcls = g_v[r + 5]
                    iw = jnp.maximum(
                        jnp.minimum(ax2, bx2) - jnp.maximum(ax1, bx1), 0.0)
                    ih = jnp.maximum(
                        jnp.minimum(ay2, by2) - jnp.maximum(ay1, by1), 0.0)
                    inter = iw * ih
                    iou = inter / (area_a + barea - inter)
                    sel = iou > best
                    best = jnp.where(sel, iou, best)
                    mx1 = jnp.where(sel, bx1, mx1)
                    my1 = jnp.where(sel, by1, my1)
                    mx2 = jnp.where(sel, bx2, mx2)
                    my2 = jnp.where(sel, by2, my2)
                    mcls = jnp.where(sel, bcls, mcls)
                    return best, mx1, my1, mx2, my2, mcls

                z = jnp.zeros((16,), jnp.float32)
                best, gx1, gy1, gx2, gy2, gcls = lax.fori_loop(
                    0, _NBOX, one_gt,
                    (jnp.full((16,), -1.0, jnp.float32), z, z, z, z, z))

                gw = gx2 - gx1
                gh = gy2 - gy1
                gcx = gx1 + gw * 0.5
                gcy = gy1 + gh * 0.5
                tx = ((gcx - acx) / aw) / 0.1
                ty = ((gcy - acy) / ah) / 0.1
                tw = _log_poly(gw / aw) / 0.2
                th = _log_poly(gh / ah) / 0.2
                pos = best >= 0.5
                ign = jnp.logical_and(best >= 0.4, best < 0.5)
                cls = jnp.where(pos, gcls, -1.0)
                cls = jnp.where(ign, -2.0, cls)

                o_v[0, pl.ds(off, 16)] = tx
                o_v[1, pl.ds(off, 16)] = ty
                o_v[2, pl.ds(off, 16)] = tw
                o_v[3, pl.ds(off, 16)] = th
                o_v[4, pl.ds(off, 16)] = cls
                o_v[5, pl.ds(off, 16)] = cls
                o_v[6, pl.ds(off, 16)] = cls
                o_v[7, pl.ds(off, 16)] = cls

            pl.loop(0, ngrp)(one_group)
            pltpu.sync_copy(o_v, out_hbm.at[b, :, pl.ds(base, APW)])

        pl.loop(0, B)(one_batch)

    return k(aT, gt_flat)


@functools.partial(jax.jit, static_argnums=())
def kernel(images, gt_boxes, gt_classes, anchor_boxes):
    del images
    B, N = gt_classes.shape
    A = anchor_boxes.shape[0]
    A_pad = 512 * ((A + 511) // 512)

    x1, y1, x2, y2 = (anchor_boxes[:, i] for i in range(4))
    aw = x2 - x1
    ah = y2 - y1
    acx = x1 + aw * 0.5
    acy = y1 + ah * 0.5
    aT = jnp.stack([x1, y1, x2, y2, aw, ah, acx, acy], axis=0)
    pad = jnp.broadcast_to(
        jnp.asarray([0.0, 0.0, 1.0, 1.0, 1.0, 1.0, 0.5, 0.5],
                    jnp.float32)[:, None], (8, A_pad - A))
    aT = jnp.concatenate([aT, pad], axis=1)

    gx1, gy1, gx2, gy2 = (gt_boxes[..., i] for i in range(4))
    area = (gx2 - gx1) * (gy2 - gy1)
    zeros = jnp.zeros_like(gx1)
    rec = jnp.stack([gx1, gy1, gx2, gy2, area, gt_classes, zeros, zeros],
                    axis=-1)
    gt_flat = rec.reshape(B, N * _GSTRIDE)
    gt_flat = jnp.repeat(gt_flat[..., None], 16, axis=-1)

    out = _sc_call(aT, gt_flat, B, A_pad)
    box = jnp.transpose(out[:, 0:4, :A], (0, 2, 1))
    cls = out[:, 4, :A]
    return box, cls


# speedup vs baseline: 2.4823x; 2.4823x over previous
import functools

import jax
import jax.numpy as jnp
from jax import lax
from jax.experimental import pallas as pl
from jax.experimental.pallas import tpu as pltpu, tpu_sc as plsc

_L = 2048
_NPAD = 104
_NBOX = 100
_GSTRIDE = 8
_LN2 = 0.6931471805599453
_A_TC = 36864


def _encode_kernel(a_ref, g_ref, gt_ref, o_ref):
    a = a_ref[...]
    ax1 = a[0:1, :]
    ay1 = a[1:2, :]
    ax2 = a[2:3, :]
    ay2 = a[3:4, :]
    aw = a[4:5, :]
    ah = a[5:6, :]
    acx = a[6:7, :]
    acy = a[7:8, :]

    g = g_ref[0]
    bx1 = g[:, 0:1]
    by1 = g[:, 1:2]
    bx2 = g[:, 2:3]
    by2 = g[:, 3:4]
    barea = g[:, 4:5]

    iw = jnp.maximum(jnp.minimum(ax2, bx2) - jnp.maximum(ax1, bx1), 0.0)
    ih = jnp.maximum(jnp.minimum(ay2, by2) - jnp.maximum(ay1, by1), 0.0)
    inter = iw * ih
    area_a = aw * ah
    union = area_a + barea - inter
    iou = inter / union

    max_iou = jnp.max(iou, axis=0, keepdims=True)
    sub = jax.lax.broadcasted_iota(jnp.int32, iou.shape, 0)
    midx = jnp.min(jnp.where(iou == max_iou, sub, _NPAD), axis=0,
                   keepdims=True)
    onehot = (sub == midx).astype(jnp.bfloat16)

    gtr = gt_ref[0]
    hi_f = jax.lax.bitcast_convert_type(
        jax.lax.bitcast_convert_type(gtr, jnp.uint32) & jnp.uint32(0xFFFF0000),
        jnp.float32)
    r1 = gtr - hi_f
    mid_f = jax.lax.bitcast_convert_type(
        jax.lax.bitcast_convert_type(r1, jnp.uint32) & jnp.uint32(0xFFFF0000),
        jnp.float32)
    lo = r1 - mid_f
    gt24 = jnp.concatenate(
        [hi_f.astype(jnp.bfloat16), mid_f.astype(jnp.bfloat16),
         lo.astype(jnp.bfloat16)], axis=0)
    g24 = jax.lax.dot_general(
        gt24, onehot, (((1,), (0,)), ((), ())),
        preferred_element_type=jnp.float32)
    g8 = (g24[0:8, :] + g24[8:16, :]) + g24[16:24, :]
    gx1 = g8[0:1, :]
    gy1 = g8[1:2, :]
    gx2 = g8[2:3, :]
    gy2 = g8[3:4, :]
    gcls = g8[5:6, :]

    gw = gx2 - gx1
    gh = gy2 - gy1
    gcx = gx1 + gw * 0.5
    gcy = gy1 + gh * 0.5

    tx = ((gcx - acx) / aw) / 0.1
    ty = ((gcy - acy) / ah) / 0.1
    tw = jnp.log(gw / aw) / 0.2
    th = jnp.log(gh / ah) / 0.2

    pos = max_iou >= 0.5
    ign = jnp.logical_and(max_iou >= 0.4, max_iou < 0.5)
    cls = jnp.where(pos, gcls, -1.0)
    cls = jnp.where(ign, -2.0, cls)

    out = jnp.concatenate(
        [tx, ty, tw, th, cls, cls, cls, cls], axis=0)
    out = jnp.where(jnp.isnan(out), -2.0, out)
    o_ref[0] = out


def _log_poly(x):
    bits = lax.bitcast_convert_type(x, jnp.int32)
    e = ((bits >> 23) & 0xFF) - 127
    m = lax.bitcast_convert_type(
        (bits & 0x7FFFFF) | 0x3F800000, jnp.float32)
    s = (m - 1.0) / (m + 1.0)
    s2 = s * s
    p = 1.0 / 11.0
    for c in (1.0 / 9.0, 1.0 / 7.0, 1.0 / 5.0, 1.0 / 3.0, 1.0):
        p = p * s2 + c
    return e.astype(jnp.float32) * _LN2 + 2.0 * s * p


def _sc_call(aT, gt_flat, B, A_pad):
    info = plsc.get_sparse_core_info()
    NC, NS = info.num_cores, info.num_subcores
    NW = NC * NS
    APW = A_pad // NW
    ngrp = APW // 16
    glen = gt_flat.shape[1]

    mesh = plsc.VectorSubcoreMesh(core_axis_name="c", subcore_axis_name="s")

    @functools.partial(
        pl.kernel, mesh=mesh,
        out_type=jax.ShapeDtypeStruct((B, 8, A_pad), jnp.float32),
        scratch_types=[
            pltpu.VMEM((8, APW), jnp.float32),
            pltpu.VMEM((glen, 16), jnp.float32),
            pltpu.VMEM((8, APW), jnp.float32),
            pltpu.SemaphoreType.DMA,
        ],
    )
    def k(aT_hbm, gt_hbm, out_hbm, a_v, g_v, o_v, sem):
        wid = lax.axis_index("s") * NC + lax.axis_index("c")
        base = wid * APW
        pltpu.sync_copy(aT_hbm.at[:, pl.ds(base, APW)], a_v)

        def one_batch(b):
            pltpu.sync_copy(gt_hbm.at[b], g_v)

            def one_group(t):
                off = t * 16
                ax1 = a_v[0, pl.ds(off, 16)]
                ay1 = a_v[1, pl.ds(off, 16)]
                ax2 = a_v[2, pl.ds(off, 16)]
                ay2 = a_v[3, pl.ds(off, 16)]
                aw = a_v[4, pl.ds(off, 16)]
                ah = a_v[5, pl.ds(off, 16)]
                acx = a_v[6, pl.ds(off, 16)]
                acy = a_v[7, pl.ds(off, 16)]
                area_a = aw * ah

                def one_gt(j, carry):
                    best, mx1, my1, mx2, my2, mcls = carry
                    r = j * _GSTRIDE
                    bx1 = g_v[r]
                    by1 = g_v[r + 1]
                    bx2 = g_v[r + 2]
                    by2 = g_v[r + 3]
                    barea = g_v[r + 4]
                    bcls = g_v[r + 5]
                    iw = jnp.maximum(
                        jnp.minimum(ax2, bx2) - jnp.maximum(ax1, bx1), 0.0)
                    ih = jnp.maximum(
                        jnp.minimum(ay2, by2) - jnp.maximum(ay1, by1), 0.0)
                    inter = iw * ih
                    iou = inter / (area_a + barea - inter)
                    sel = iou > best
                    best = jnp.where(sel, iou, best)
                    mx1 = jnp.where(sel, bx1, mx1)
                    my1 = jnp.where(sel, by1, my1)
                    mx2 = jnp.where(sel, bx2, mx2)
                    my2 = jnp.where(sel, by2, my2)
                    mcls = jnp.where(sel, bcls, mcls)
                    return best, mx1, my1, mx2, my2, mcls

                z = jnp.zeros((16,), jnp.float32)
                best, gx1, gy1, gx2, gy2, gcls = lax.fori_loop(
                    0, _NBOX, one_gt,
                    (jnp.full((16,), -1.0, jnp.float32), z, z, z, z, z))

                gw = gx2 - gx1
                gh = gy2 - gy1
                gcx = gx1 + gw * 0.5
                gcy = gy1 + gh * 0.5
                tx = ((gcx - acx) / aw) / 0.1
                ty = ((gcy - acy) / ah) / 0.1
                tw = _log_poly(gw / aw) / 0.2
                th = _log_poly(gh / ah) / 0.2
                pos = best >= 0.5
                ign = jnp.logical_and(best >= 0.4, best < 0.5)
                cls = jnp.where(pos, gcls, -1.0)
                cls = jnp.where(ign, -2.0, cls)

                o_v[0, pl.ds(off, 16)] = tx
                o_v[1, pl.ds(off, 16)] = ty
                o_v[2, pl.ds(off, 16)] = tw
                o_v[3, pl.ds(off, 16)] = th
                o_v[4, pl.ds(off, 16)] = cls
                o_v[5, pl.ds(off, 16)] = cls
                o_v[6, pl.ds(off, 16)] = cls
                o_v[7, pl.ds(off, 16)] = cls

            pl.loop(0, ngrp)(one_group)
            pltpu.sync_copy(o_v, out_hbm.at[b, :, pl.ds(base, APW)])

        pl.loop(0, B)(one_batch)

    return k(aT, gt_flat)


def _tc_call(aT, gt_cols, gt_rowsT, B, A_tc):
    G = A_tc // _L
    return pl.pallas_call(
        _encode_kernel,
        grid=(G, B),
        in_specs=[
            pl.BlockSpec((8, _L), lambda g, b: (0, g)),
            pl.BlockSpec((1, _NPAD, 8), lambda g, b: (b, 0, 0)),
            pl.BlockSpec((1, 8, _NPAD), lambda g, b: (b, 0, 0)),
        ],
        out_specs=pl.BlockSpec((1, 8, _L), lambda g, b: (b, 0, g)),
        out_shape=jax.ShapeDtypeStruct((B, 8, A_tc), jnp.float32),
    )(aT, gt_cols, gt_rowsT)


@functools.partial(jax.jit, static_argnums=())
def kernel(images, gt_boxes, gt_classes, anchor_boxes):
    del images
    B, N = gt_classes.shape
    A = anchor_boxes.shape[0]
    A_sc_real = A - _A_TC
    A_sc = 512 * ((A_sc_real + 511) // 512)

    x1, y1, x2, y2 = (anchor_boxes[:, i] for i in range(4))
    aw = x2 - x1
    ah = y2 - y1
    acx = x1 + aw * 0.5
    acy = y1 + ah * 0.5
    aT = jnp.stack([x1, y1, x2, y2, aw, ah, acx, acy], axis=0)
    pad = jnp.broadcast_to(
        jnp.asarray([0.0, 0.0, 1.0, 1.0, 1.0, 1.0, 0.5, 0.5],
                    jnp.float32)[:, None], (8, _A_TC + A_sc - A))
    aT_tc = aT[:, :_A_TC]
    aT_sc = jnp.concatenate([aT[:, _A_TC:], pad], axis=1)

    gx1, gy1, gx2, gy2 = (gt_boxes[..., i] for i in range(4))
    area = (gx2 - gx1) * (gy2 - gy1)
    zeros = jnp.zeros_like(gx1)
    cols = jnp.stack([gx1, gy1, gx2, gy2, area, gt_classes, zeros, zeros],
                     axis=-1)
    gt_cols = jnp.pad(cols, ((0, 0), (0, _NPAD - N), (0, 0)))
    gt_rowsT = jnp.transpose(gt_cols, (0, 2, 1))
    gt_flat = cols.reshape(B, N * _GSTRIDE)
    gt_splat = jnp.repeat(gt_flat[..., None], 16, axis=-1)

    out_sc = _sc_call(aT_sc, gt_splat, B, A_sc)
    out_tc = _tc_call(aT_tc, gt_cols, gt_rowsT, B, _A_TC)

    box = jnp.concatenate(
        [jnp.transpose(out_tc[:, 0:4, :], (0, 2, 1)),
         jnp.transpose(out_sc[:, 0:4, :A_sc_real], (0, 2, 1))], axis=1)
    cls = jnp.concatenate(
        [out_tc[:, 4, :], out_sc[:, 4, :A_sc_real]], axis=1)
    return box, cls


# speedup vs baseline: 2.5025x; 1.0081x over previous
import functools

import jax
import jax.numpy as jnp
from jax import lax
from jax.experimental import pallas as pl
from jax.experimental.pallas import tpu as pltpu, tpu_sc as plsc

_L = 2048
_NPAD = 104
_NBOX = 100
_GSTRIDE = 8
_LN2 = 0.6931471805599453
_A_TC = 40960


def _encode_kernel(a_ref, g_ref, gt_ref, o_ref):
    a = a_ref[...]
    ax1 = a[0:1, :]
    ay1 = a[1:2, :]
    ax2 = a[2:3, :]
    ay2 = a[3:4, :]
    aw = a[4:5, :]
    ah = a[5:6, :]
    acx = a[6:7, :]
    acy = a[7:8, :]

    g = g_ref[0]
    bx1 = g[:, 0:1]
    by1 = g[:, 1:2]
    bx2 = g[:, 2:3]
    by2 = g[:, 3:4]
    barea = g[:, 4:5]

    iw = jnp.maximum(jnp.minimum(ax2, bx2) - jnp.maximum(ax1, bx1), 0.0)
    ih = jnp.maximum(jnp.minimum(ay2, by2) - jnp.maximum(ay1, by1), 0.0)
    inter = iw * ih
    area_a = aw * ah
    union = area_a + barea - inter
    iou = inter / union

    max_iou = jnp.max(iou, axis=0, keepdims=True)
    sub = jax.lax.broadcasted_iota(jnp.int32, iou.shape, 0)
    midx = jnp.min(jnp.where(iou == max_iou, sub, _NPAD), axis=0,
                   keepdims=True)
    onehot = (sub == midx).astype(jnp.bfloat16)

    gtr = gt_ref[0]
    hi_f = jax.lax.bitcast_convert_type(
        jax.lax.bitcast_convert_type(gtr, jnp.uint32) & jnp.uint32(0xFFFF0000),
        jnp.float32)
    r1 = gtr - hi_f
    mid_f = jax.lax.bitcast_convert_type(
        jax.lax.bitcast_convert_type(r1, jnp.uint32) & jnp.uint32(0xFFFF0000),
        jnp.float32)
    lo = r1 - mid_f
    gt24 = jnp.concatenate(
        [hi_f.astype(jnp.bfloat16), mid_f.astype(jnp.bfloat16),
         lo.astype(jnp.bfloat16)], axis=0)
    g24 = jax.lax.dot_general(
        gt24, onehot, (((1,), (0,)), ((), ())),
        preferred_element_type=jnp.float32)
    g8 = (g24[0:8, :] + g24[8:16, :]) + g24[16:24, :]
    gx1 = g8[0:1, :]
    gy1 = g8[1:2, :]
    gx2 = g8[2:3, :]
    gy2 = g8[3:4, :]
    gcls = g8[5:6, :]

    gw = gx2 - gx1
    gh = gy2 - gy1
    gcx = gx1 + gw * 0.5
    gcy = gy1 + gh * 0.5

    tx = ((gcx - acx) / aw) / 0.1
    ty = ((gcy - acy) / ah) / 0.1
    tw = jnp.log(gw / aw) / 0.2
    th = jnp.log(gh / ah) / 0.2

    pos = max_iou >= 0.5
    ign = jnp.logical_and(max_iou >= 0.4, max_iou < 0.5)
    cls = jnp.where(pos, gcls, -1.0)
    cls = jnp.where(ign, -2.0, cls)

    out = jnp.concatenate(
        [tx, ty, tw, th, cls, cls, cls, cls], axis=0)
    out = jnp.where(jnp.isnan(out), -2.0, out)
    o_ref[0] = out


def _log_poly(x):
    bits = lax.bitcast_convert_type(x, jnp.int32)
    e = ((bits >> 23) & 0xFF) - 127
    m = lax.bitcast_convert_type(
        (bits & 0x7FFFFF) | 0x3F800000, jnp.float32)
    s = (m - 1.0) / (m + 1.0)
    s2 = s * s
    p = 1.0 / 11.0
    for c in (1.0 / 9.0, 1.0 / 7.0, 1.0 / 5.0, 1.0 / 3.0, 1.0):
        p = p * s2 + c
    return e.astype(jnp.float32) * _LN2 + 2.0 * s * p


def _sc_call(aT, gt_flat, B, A_pad):
    info = plsc.get_sparse_core_info()
    NC, NS = info.num_cores, info.num_subcores
    NW = NC * NS
    APW = A_pad // NW
    ngrp = APW // 16
    glen = gt_flat.shape[1]

    mesh = plsc.VectorSubcoreMesh(core_axis_name="c", subcore_axis_name="s")

    @functools.partial(
        pl.kernel, mesh=mesh,
        out_type=jax.ShapeDtypeStruct((B, 8, A_pad), jnp.float32),
        scratch_types=[
            pltpu.VMEM((8, APW), jnp.float32),
            pltpu.VMEM((glen, 16), jnp.float32),
            pltpu.VMEM((8, APW), jnp.float32),
            pltpu.SemaphoreType.DMA,
        ],
    )
    def k(aT_hbm, gt_hbm, out_hbm, a_v, g_v, o_v, sem):
        wid = lax.axis_index("s") * NC + lax.axis_index("c")
        base = wid * APW
        pltpu.sync_copy(aT_hbm.at[:, pl.ds(base, APW)], a_v)

        def one_batch(b):
            pltpu.sync_copy(gt_hbm.at[b], g_v)

            def one_group(t):
                off = t * 16
                ax1 = a_v[0, pl.ds(off, 16)]
                ay1 = a_v[1, pl.ds(off, 16)]
                ax2 = a_v[2, pl.ds(off, 16)]
                ay2 = a_v[3, pl.ds(off, 16)]
                aw = a_v[4, pl.ds(off, 16)]
                ah = a_v[5, pl.ds(off, 16)]
                acx = a_v[6, pl.ds(off, 16)]
                acy = a_v[7, pl.ds(off, 16)]
                area_a = aw * ah

                def one_gt(j, carry):
                    best, mx1, my1, mx2, my2, mcls = carry
                    r = j * _GSTRIDE
                    bx1 = g_v[r]
                    by1 = g_v[r + 1]
                    bx2 = g_v[r + 2]
                    by2 = g_v[r + 3]
                    barea = g_v[r + 4]
                    bcls = g_v[r + 5]
                    iw = jnp.maximum(
                        jnp.minimum(ax2, bx2) - jnp.maximum(ax1, bx1), 0.0)
                    ih = jnp.maximum(
                        jnp.minimum(ay2, by2) - jnp.maximum(ay1, by1), 0.0)
                    inter = iw * ih
                    iou = inter / (area_a + barea - inter)
                    sel = iou > best
                    best = jnp.where(sel, iou, best)
                    mx1 = jnp.where(sel, bx1, mx1)
                    my1 = jnp.where(sel, by1, my1)
                    mx2 = jnp.where(sel, bx2, mx2)
                    my2 = jnp.where(sel, by2, my2)
                    mcls = jnp.where(sel, bcls, mcls)
                    return best, mx1, my1, mx2, my2, mcls

                z = jnp.zeros((16,), jnp.float32)
                best, gx1, gy1, gx2, gy2, gcls = lax.fori_loop(
                    0, _NBOX, one_gt,
                    (jnp.full((16,), -1.0, jnp.float32), z, z, z, z, z))

                gw = gx2 - gx1
                gh = gy2 - gy1
                gcx = gx1 + gw * 0.5
                gcy = gy1 + gh * 0.5
                tx = ((gcx - acx) / aw) / 0.1
                ty = ((gcy - acy) / ah) / 0.1
                tw = _log_poly(gw / aw) / 0.2
                th = _log_poly(gh / ah) / 0.2
                pos = best >= 0.5
                ign = jnp.logical_and(best >= 0.4, best < 0.5)
                cls = jnp.where(pos, gcls, -1.0)
                cls = jnp.where(ign, -2.0, cls)

                o_v[0, pl.ds(off, 16)] = tx
                o_v[1, pl.ds(off, 16)] = ty
                o_v[2, pl.ds(off, 16)] = tw
                o_v[3, pl.ds(off, 16)] = th
                o_v[4, pl.ds(off, 16)] = cls
                o_v[5, pl.ds(off, 16)] = cls
                o_v[6, pl.ds(off, 16)] = cls
                o_v[7, pl.ds(off, 16)] = cls

            pl.loop(0, ngrp)(one_group)
            pltpu.sync_copy(o_v, out_hbm.at[b, :, pl.ds(base, APW)])

        pl.loop(0, B)(one_batch)

    return k(aT, gt_flat)


def _tc_call(aT, gt_cols, gt_rowsT, B, A_tc):
    G = A_tc // _L
    return pl.pallas_call(
        _encode_kernel,
        grid=(G, B),
        in_specs=[
            pl.BlockSpec((8, _L), lambda g, b: (0, g)),
            pl.BlockSpec((1, _NPAD, 8), lambda g, b: (b, 0, 0)),
            pl.BlockSpec((1, 8, _NPAD), lambda g, b: (b, 0, 0)),
        ],
        out_specs=pl.BlockSpec((1, 8, _L), lambda g, b: (b, 0, g)),
        out_shape=jax.ShapeDtypeStruct((B, 8, A_tc), jnp.float32),
    )(aT, gt_cols, gt_rowsT)


@functools.partial(jax.jit, static_argnums=())
def kernel(images, gt_boxes, gt_classes, anchor_boxes):
    del images
    B, N = gt_classes.shape
    A = anchor_boxes.shape[0]
    A_sc_real = A - _A_TC
    A_sc = 512 * ((A_sc_real + 511) // 512)

    x1, y1, x2, y2 = (anchor_boxes[:, i] for i in range(4))
    aw = x2 - x1
    ah = y2 - y1
    acx = x1 + aw * 0.5
    acy = y1 + ah * 0.5
    aT = jnp.stack([x1, y1, x2, y2, aw, ah, acx, acy], axis=0)
    pad = jnp.broadcast_to(
        jnp.asarray([0.0, 0.0, 1.0, 1.0, 1.0, 1.0, 0.5, 0.5],
                    jnp.float32)[:, None], (8, _A_TC + A_sc - A))
    aT_tc = aT[:, :_A_TC]
    aT_sc = jnp.concatenate([aT[:, _A_TC:], pad], axis=1)

    gx1, gy1, gx2, gy2 = (gt_boxes[..., i] for i in range(4))
    area = (gx2 - gx1) * (gy2 - gy1)
    zeros = jnp.zeros_like(gx1)
    cols = jnp.stack([gx1, gy1, gx2, gy2, area, gt_classes, zeros, zeros],
                     axis=-1)
    gt_cols = jnp.pad(cols, ((0, 0), (0, _NPAD - N), (0, 0)))
    gt_rowsT = jnp.transpose(gt_cols, (0, 2, 1))
    gt_flat = cols.reshape(B, N * _GSTRIDE)
    gt_splat = jnp.repeat(gt_flat[..., None], 16, axis=-1)

    out_sc = _sc_call(aT_sc, gt_splat, B, A_sc)
    out_tc = _tc_call(aT_tc, gt_cols, gt_rowsT, B, _A_TC)

    box = jnp.concatenate(
        [jnp.transpose(out_tc[:, 0:4, :], (0, 2, 1)),
         jnp.transpose(out_sc[:, 0:4, :A_sc_real], (0, 2, 1))], axis=1)
    cls = jnp.concatenate(
        [out_tc[:, 4, :], out_sc[:, 4, :A_sc_real]], axis=1)
    return box, cls
